# Initial kernel scaffold; baseline (speedup 1.0000x reference)
import jax
import jax.numpy as jnp
from jax.experimental import pallas as pl


def kernel(h, edge_index, eps, W, b, loc, log_scale):
    raise NotImplementedError("write your pallas kernel here")


# speedup vs baseline: 21.5604x; 21.5604x over previous
import jax
import jax.numpy as jnp
from jax import lax
from jax.experimental import pallas as pl
from jax.experimental.pallas import tpu as pltpu
from jax.experimental.pallas import tpu_sc as plsc

NC = 2
NS = 16
NW = NC * NS
LANES = 16

N = 10000
E = 320000
D = 128

EPW = E // NW
C = 80
NCH = EPW // C
PS = 24
NPASS = NCH // PS
TAIL = NCH - NPASS * PS
DP = D // 2
RPT = 640
ZR = 80


def _hist_body(src_hbm, dst_hbm, degs_out, degd_out, src_v, dst_v, degs_v, degd_v):
    c = lax.axis_index("c")
    s = lax.axis_index("s")
    w = c * NS + s
    pltpu.sync_copy(src_hbm.at[w], src_v)
    pltpu.sync_copy(dst_hbm.at[w], dst_v)
    z = jnp.zeros((LANES,), jnp.float32)
    ones = jnp.ones((LANES,), jnp.float32)

    def zero_body(i, _):
        degs_v[pl.ds(i * LANES, LANES)] = z
        degd_v[pl.ds(i * LANES, LANES)] = z
        return 0

    lax.fori_loop(0, N // LANES, zero_body, 0)

    def hist_row(i, _):
        for k in range(C // LANES):
            sl = pl.ds(k * LANES, LANES)
            plsc.addupdate_scatter(degs_v, [src_v[i, sl]], ones)
            plsc.addupdate_scatter(degd_v, [dst_v[i, sl]], ones)
        return 0

    lax.fori_loop(0, NCH, hist_row, 0)
    pltpu.sync_copy(degs_v, degs_out.at[w])
    pltpu.sync_copy(degd_v, degd_out.at[w])


PSE = PS * C


def _edge_body(hsc_hbm, src_hbm, dst_hbm, eps_hbm, loc_hbm, lsc_hbm, part_out,
               src_v, dst_v, eps_v, ibuf_v, rows_v, loc_v, lsc_v, agg_sh,
               sem0, sem1, ssem0, ssem1):
    c = lax.axis_index("c")
    s = lax.axis_index("s")
    w = c * NS + s
    pltpu.sync_copy(loc_hbm, loc_v)
    pltpu.sync_copy(lsc_hbm, lsc_v)
    loc = loc_v[...]
    scale = jnp.exp(lsc_v[...])

    zrow = jnp.zeros((LANES,), jnp.float32)

    def zb_row(i, _):
        for k in range(D // LANES):
            rows_v[0, i, pl.ds(k * LANES, LANES)] = zrow
        return 0

    lax.fori_loop(0, ZR, zb_row, 0)
    base = s * RPT
    nco = jnp.where(s == NS - 1, (N - (NS - 1) * RPT) // ZR, RPT // ZR)

    def zcopy(k, _):
        pltpu.sync_copy(rows_v.at[0], agg_sh.at[pl.ds(base + k * ZR, ZR)])
        return 0

    lax.fori_loop(0, nco, zcopy, 0)
    plsc.subcore_barrier()

    def do_pass(p_chunk0, nchunks):
        e0 = w * EPW + p_chunk0 * C
        ne = nchunks * C
        pltpu.sync_copy(src_hbm.at[pl.ds(e0, ne)], src_v.at[pl.ds(0, ne)])
        pltpu.sync_copy(dst_hbm.at[w, pl.ds(p_chunk0, nchunks)],
                        dst_v.at[pl.ds(0, nchunks)])
        pltpu.sync_copy(eps_hbm.at[pl.ds(e0, ne)], eps_v.at[pl.ds(0, ne)])

        def coef_vec(m, _):
            sl = pl.ds(m * LANES, LANES)
            zz = loc + scale * eps_v[sl]
            eps_v[sl] = 1.0 / (1.0 + jnp.exp(-zz))
            return 0

        lax.fori_loop(0, nchunks * (C // LANES), coef_vec, 0)

        def scale_rows(j, b):
            for g in range(C // LANES):
                coefs = eps_v[pl.ds(j * C + g * LANES, LANES)]
                for t in range(LANES):
                    cf = coefs[t]
                    i = g * LANES + t
                    for k in range(DP // LANES):
                        word = ibuf_v[b, i, pl.ds(k * LANES, LANES)]
                        a, b2 = plsc.unpack(
                            plsc.bitcast(word, jnp.bfloat16),
                            format=plsc.PackFormat.INTERLEAVED)
                        rows_v[b, i, pl.ds(k * LANES, LANES)] = a * cf
                        rows_v[b, i, pl.ds(DP + k * LANES, LANES)] = b2 * cf

        def start_gather(j, b, sem):
            pltpu.async_copy(hsc_hbm.at[src_v.at[pl.ds(j * C, C)]],
                             ibuf_v.at[b], sem)

        def wait_gather(b, sem):
            pltpu.make_async_copy(hsc_hbm.at[src_v.at[pl.ds(0, C)]],
                                  ibuf_v.at[b], sem).wait()

        def start_scatter(j, b, sem):
            pltpu.async_copy(rows_v.at[b], agg_sh.at[dst_v.at[j]], sem,
                             add=True)

        def wait_scatter(j, b, sem):
            pltpu.make_async_copy(rows_v.at[b], agg_sh.at[dst_v.at[j]],
                                  sem).wait()

        if nchunks % 2 == 0:
            start_gather(0, 0, sem0)
            start_gather(1, 1, sem1)

            def pair(jj, _):
                j = jj * 2
                wait_gather(0, sem0)

                @pl.when(jj > 0)
                def _():
                    wait_scatter(j - 2, 0, ssem0)

                scale_rows(j, 0)

                @pl.when(j + 2 < nchunks)
                def _():
                    start_gather(j + 2, 0, sem0)

                start_scatter(j, 0, ssem0)
                wait_gather(1, sem1)

                @pl.when(jj > 0)
                def _():
                    wait_scatter(j - 1, 1, ssem1)

                scale_rows(j + 1, 1)

                @pl.when(j + 3 < nchunks)
                def _():
                    start_gather(j + 3, 1, sem1)

                start_scatter(j + 1, 1, ssem1)
                return 0

            lax.fori_loop(0, nchunks // 2, pair, 0)
            wait_scatter(nchunks - 2, 0, ssem0)
            wait_scatter(nchunks - 1, 1, ssem1)
        else:
            def chunk(j, _):
                start_gather(j, 0, sem0)
                wait_gather(0, sem0)
                scale_rows(j, 0)
                pltpu.sync_copy(rows_v.at[0], agg_sh.at[dst_v.at[j]],
                                add=True)
                return 0

            lax.fori_loop(0, nchunks, chunk, 0)

    def pass_body(p, _):
        do_pass(pl.multiple_of(p * PS, 8), PS)
        return 0

    lax.fori_loop(0, NPASS, pass_body, 0)
    do_pass(NPASS * PS, TAIL)
    plsc.subcore_barrier()

    def dump(k, _):
        r0 = base + k * ZR
        pltpu.sync_copy(agg_sh.at[pl.ds(r0, ZR)], part_out.at[c, pl.ds(r0, ZR)])
        return 0

    lax.fori_loop(0, nco, dump, 0)


_sc_mesh = plsc.VectorSubcoreMesh(
    core_axis_name="c", subcore_axis_name="s", num_cores=NC, num_subcores=NS)

_sc_params = pltpu.CompilerParams(needs_layout_passes=False)

_hist_call = pl.kernel(
    _hist_body,
    out_type=(jax.ShapeDtypeStruct((NW, N), jnp.float32),
              jax.ShapeDtypeStruct((NW, N), jnp.float32)),
    mesh=_sc_mesh,
    scratch_types=[
        pltpu.VMEM((NCH, C), jnp.int32),
        pltpu.VMEM((NCH, C), jnp.int32),
        pltpu.VMEM((N,), jnp.float32),
        pltpu.VMEM((N,), jnp.float32),
    ],
    compiler_params=_sc_params,
)

_edge_call = pl.kernel(
    _edge_body,
    out_type=jax.ShapeDtypeStruct((NC, N, D), jnp.float32),
    mesh=_sc_mesh,
    scratch_types=[
        pltpu.VMEM((PSE,), jnp.int32),
        pltpu.VMEM((PS, C), jnp.int32),
        pltpu.VMEM((PSE,), jnp.float32),
        pltpu.VMEM((2, C, DP), jnp.int32),
        pltpu.VMEM((2, C, D), jnp.float32),
        pltpu.VMEM((LANES,), jnp.float32),
        pltpu.VMEM((LANES,), jnp.float32),
        pltpu.VMEM_SHARED((N, D), jnp.float32),
        pltpu.SemaphoreType.DMA,
        pltpu.SemaphoreType.DMA,
        pltpu.SemaphoreType.DMA,
        pltpu.SemaphoreType.DMA,
    ],
    compiler_params=pltpu.CompilerParams(
        needs_layout_passes=False, use_tc_tiling_on_sc=False),
)

RB = 512
_GRID = (N + RB - 1) // RB


def _mm_body(h_ref, w_ref, degs_ref, out_ref):
    deg = jnp.sum(degs_ref[...], axis=0)
    norm = lax.rsqrt(jnp.where(deg > 0, deg, 1.0))
    hs = jnp.dot(h_ref[...] * norm[:, None], w_ref[...],
                 preferred_element_type=jnp.float32)
    lo = lax.bitcast_convert_type(hs[:, :DP], jnp.int32)
    hi = lax.bitcast_convert_type(hs[:, DP:], jnp.int32)
    rlo = lo + 0x7FFF + ((lo >> 16) & 1)
    rhi = hi + 0x7FFF + ((hi >> 16) & 1)
    out_ref[...] = (rhi & jnp.int32(-65536)) | ((rlo >> 16) & 0xFFFF)


def _fin_body(p_ref, degd_ref, b_ref, out_ref):
    deg = jnp.sum(degd_ref[...], axis=0)
    norm = lax.rsqrt(jnp.where(deg > 0, deg, 1.0))
    out_ref[...] = (p_ref[0] + p_ref[1]) * norm[:, None] + b_ref[...]


def kernel(h, edge_index, eps, W, b, loc, log_scale):
    src3 = edge_index[0].reshape(NW, NCH, C)
    dst3 = edge_index[1].reshape(NW, NCH, C)
    loc16 = jnp.broadcast_to(loc, (LANES,))
    ls16 = jnp.broadcast_to(log_scale, (LANES,))

    degs, degd = _hist_call(src3, dst3)

    h_scaled = pl.pallas_call(
        _mm_body,
        grid=(_GRID,),
        in_specs=[
            pl.BlockSpec((RB, D), lambda r: (r, 0)),
            pl.BlockSpec((D, D), lambda r: (0, 0)),
            pl.BlockSpec((NW, RB), lambda r: (0, r)),
        ],
        out_specs=pl.BlockSpec((RB, DP), lambda r: (r, 0)),
        out_shape=jax.ShapeDtypeStruct((N, DP), jnp.int32),
    )(h, W, degs)

    partials = _edge_call(h_scaled, edge_index[0], dst3, eps.reshape(E),
                          loc16, ls16)

    out = pl.pallas_call(
        _fin_body,
        grid=(_GRID,),
        in_specs=[
            pl.BlockSpec((NC, RB, D), lambda r: (0, r, 0)),
            pl.BlockSpec((NW, RB), lambda r: (0, r)),
            pl.BlockSpec((1, D), lambda r: (0, 0)),
        ],
        out_specs=pl.BlockSpec((RB, D), lambda r: (r, 0)),
        out_shape=jax.ShapeDtypeStruct((N, D), jnp.float32),
    )(partials, degd, b.reshape(1, D))
    return out
